# Initial kernel scaffold; baseline (speedup 1.0000x reference)
import functools

import jax
import jax.numpy as jnp
from jax import lax
from jax.experimental import pallas as pl
from jax.experimental.pallas import tpu as pltpu
from jax.experimental.pallas import tpu_sc as plsc

VOCAB = 1000000
EMBED_DIM = 64
BATCH = 16384
SEQ = 50

NC = 2
NS = 16
NW = NC * NS

B = BATCH * SEQ
IDX_MINOR = 128
IDX_ROWS = B // IDX_MINOR
ROWS_PER_W = IDX_ROWS // NW
K = 8
CHUNK = K * IDX_MINOR
N_CHUNKS = ROWS_PER_W // K


def _gather_kernel(table_hbm, idx_hbm, out_hbm, idx_v, rows_v, gsem):
    wid = lax.axis_index("s") * NC + lax.axis_index("c")
    idx_row0 = wid * ROWS_PER_W
    out_row0 = wid * ROWS_PER_W * IDX_MINOR

    @pl.loop(0, N_CHUNKS)
    def _chunk(g):
        r = idx_row0 + g * K
        pltpu.sync_copy(idx_hbm.at[pl.ds(r, K)], idx_v)
        cps = [
            pltpu.async_copy(
                table_hbm.at[idx_v.at[j]],
                rows_v.at[pl.ds(j * IDX_MINOR, IDX_MINOR)],
                gsem,
            )
            for j in range(K)
        ]
        for cp in cps:
            cp.wait()
        pltpu.sync_copy(rows_v, out_hbm.at[pl.ds(out_row0 + g * CHUNK, CHUNK)])


@jax.jit
def _embedding_gather(x2d, table):
    mesh = plsc.VectorSubcoreMesh(core_axis_name="c", subcore_axis_name="s")
    run = functools.partial(
        pl.kernel,
        mesh=mesh,
        out_type=jax.ShapeDtypeStruct((B, EMBED_DIM), jnp.float32),
        scratch_types=[
            pltpu.VMEM((K, IDX_MINOR), jnp.int32),
            pltpu.VMEM((CHUNK, EMBED_DIM), jnp.float32),
            pltpu.SemaphoreType.DMA,
        ],
    )(_gather_kernel)
    return run(table, x2d)


def kernel(x, embedding):
    x2d = x.reshape(IDX_ROWS, IDX_MINOR).astype(jnp.int32)
    out = _embedding_gather(x2d, embedding)
    return out.reshape(BATCH, SEQ, EMBED_DIM)


# speedup vs baseline: 1.8437x; 1.8437x over previous
import functools

import jax
import jax.numpy as jnp
from jax import lax
from jax.experimental import pallas as pl
from jax.experimental.pallas import tpu as pltpu
from jax.experimental.pallas import tpu_sc as plsc

VOCAB = 1000000
EMBED_DIM = 64
BATCH = 16384
SEQ = 50

NC = 2
NS = 16
NW = NC * NS

B = BATCH * SEQ
IDX_MINOR = 128
IDX_ROWS = B // IDX_MINOR
ROWS_PER_W = IDX_ROWS // NW
K = 8
CHUNK = K * IDX_MINOR
N_CHUNKS = ROWS_PER_W // K


def _gather_kernel(table_hbm, idx_hbm, out_hbm, idx_v, rows_v, gsem):
    wid = lax.axis_index("s") * NC + lax.axis_index("c")
    idx_row0 = wid * ROWS_PER_W
    out_row0 = wid * ROWS_PER_W * IDX_MINOR

    @pl.loop(0, N_CHUNKS)
    def _chunk(g):
        r = idx_row0 + g * K
        pltpu.sync_copy(idx_hbm.at[pl.ds(r, K)], idx_v)
        cps = [
            pltpu.async_copy(
                table_hbm.at[idx_v.at[j]],
                rows_v.at[pl.ds(j * IDX_MINOR, IDX_MINOR)],
                gsem,
            )
            for j in range(K)
        ]
        for cp in cps:
            cp.wait()
        pltpu.sync_copy(rows_v, out_hbm.at[pl.ds(out_row0 + g * CHUNK, CHUNK)])


@jax.jit
def _embedding_gather(x2d, table):
    mesh = plsc.VectorSubcoreMesh(core_axis_name="c", subcore_axis_name="s")
    run = functools.partial(
        pl.kernel,
        mesh=mesh,
        out_type=jax.ShapeDtypeStruct((B, EMBED_DIM), jnp.float32),
        scratch_types=[
            pltpu.VMEM((K, IDX_MINOR), jnp.int32),
            pltpu.VMEM((CHUNK, EMBED_DIM), jnp.float32),
            pltpu.SemaphoreType.DMA,
        ],
        compiler_params=pltpu.CompilerParams(use_tc_tiling_on_sc=False),
    )(_gather_kernel)
    return run(table, x2d)


def kernel(x, embedding):
    x2d = x.reshape(IDX_ROWS, IDX_MINOR).astype(jnp.int32)
    out = _embedding_gather(x2d, embedding)
    return out.reshape(BATCH, SEQ, EMBED_DIM)


# speedup vs baseline: 1.8727x; 1.0157x over previous
import functools

import jax
import jax.numpy as jnp
from jax import lax
from jax.experimental import pallas as pl
from jax.experimental.pallas import tpu as pltpu
from jax.experimental.pallas import tpu_sc as plsc

VOCAB = 1000000
EMBED_DIM = 64
BATCH = 16384
SEQ = 50

NC = 2
NS = 16
NW = NC * NS

B = BATCH * SEQ
IDX_MINOR = 128
IDX_ROWS = B // IDX_MINOR
ROWS_PER_W = IDX_ROWS // NW
K = 4
CHUNK = K * IDX_MINOR
N_CHUNKS = ROWS_PER_W // K
NB = 2


def _gather_kernel(table, idx, out, idx_all, rows_a, rows_b,
                   gsem_a, gsem_b, wsem_a, wsem_b):
    wid = lax.axis_index("s") * NC + lax.axis_index("c")
    idx_row0 = wid * ROWS_PER_W
    out_row0 = wid * ROWS_PER_W * IDX_MINOR

    rows = (rows_a, rows_b)
    gsem = (gsem_a, gsem_b)
    wsem = (wsem_a, wsem_b)

    pltpu.sync_copy(idx.at[pl.ds(idx_row0, ROWS_PER_W)], idx_all)

    def fire_gathers(g, b):
        for j in range(K):
            pltpu.async_copy(
                table.at[idx_all.at[g * K + j]],
                rows[b].at[pl.ds(j * IDX_MINOR, IDX_MINOR)],
                gsem[b],
            )

    def drain_gathers(b):
        pltpu.make_async_copy(table.at[pl.ds(0, CHUNK)], rows[b], gsem[b]).wait()

    def fire_writeback(g, b):
        pltpu.async_copy(rows[b], out.at[pl.ds(out_row0 + g * CHUNK, CHUNK)],
                         wsem[b])

    def drain_writeback(b):
        pltpu.make_async_copy(rows[b], out.at[pl.ds(out_row0, CHUNK)],
                              wsem[b]).wait()

    for b in range(NB):
        fire_gathers(b, b)

    @pl.loop(0, N_CHUNKS, step=NB)
    def _step(g0):
        for b in range(NB):
            g = g0 + b
            drain_gathers(b)
            fire_writeback(g, b)
            nxt = g + NB

            @pl.when(g >= NB)
            def _():
                drain_writeback(b)

            @pl.when(nxt < N_CHUNKS)
            def _():
                fire_gathers(nxt, b)

    for b in range(NB):
        drain_writeback(b)


@jax.jit
def _embedding_gather(x2d, table):
    mesh = plsc.VectorSubcoreMesh(core_axis_name="c", subcore_axis_name="s")
    run = functools.partial(
        pl.kernel,
        mesh=mesh,
        out_type=jax.ShapeDtypeStruct((B, EMBED_DIM), jnp.float32),
        scratch_types=[
            pltpu.VMEM((ROWS_PER_W, IDX_MINOR), jnp.int32),
            pltpu.VMEM((CHUNK, EMBED_DIM), jnp.float32),
            pltpu.VMEM((CHUNK, EMBED_DIM), jnp.float32),
            pltpu.SemaphoreType.DMA,
            pltpu.SemaphoreType.DMA,
            pltpu.SemaphoreType.DMA,
            pltpu.SemaphoreType.DMA,
        ],
        compiler_params=pltpu.CompilerParams(use_tc_tiling_on_sc=False),
    )(_gather_kernel)
    return run(table, x2d)


def kernel(x, embedding):
    x2d = x.reshape(IDX_ROWS, IDX_MINOR).astype(jnp.int32)
    out = _embedding_gather(x2d, embedding)
    return out.reshape(BATCH, SEQ, EMBED_DIM)


# speedup vs baseline: 1.8765x; 1.0020x over previous
import functools

import jax
import jax.numpy as jnp
from jax import lax
from jax.experimental import pallas as pl
from jax.experimental.pallas import tpu as pltpu
from jax.experimental.pallas import tpu_sc as plsc

VOCAB = 1000000
EMBED_DIM = 64
BATCH = 16384
SEQ = 50

NC = 2
NS = 16
NW = NC * NS

B = BATCH * SEQ
IDX_MINOR = 128
IDX_ROWS = B // IDX_MINOR
ROWS_PER_W = IDX_ROWS // NW
K = 4
CHUNK = K * IDX_MINOR
N_CHUNKS = ROWS_PER_W // K
NB = 2


def _gather_kernel(table, idx, out, idx_all, rows_a, rows_b,
                   gsem_a, gsem_b, wsem_a, wsem_b):
    wid = lax.axis_index("s") * NC + lax.axis_index("c")
    idx_row0 = wid * ROWS_PER_W
    out_row0 = wid * ROWS_PER_W * IDX_MINOR

    rows = (rows_a, rows_b)
    gsem = (gsem_a, gsem_b)
    wsem = (wsem_a, wsem_b)

    pltpu.sync_copy(idx.at[pl.ds(idx_row0, ROWS_PER_W)], idx_all)

    def fire_gathers(g, b):
        for j in range(K):
            pltpu.async_copy(
                table.at[idx_all.at[g * K + j]],
                rows[b].at[pl.ds(j * IDX_MINOR, IDX_MINOR)],
                gsem[b],
            )

    def drain_gathers(b):
        pltpu.make_async_copy(table.at[pl.ds(0, CHUNK)], rows[b], gsem[b]).wait()

    def fire_writeback(g, b):
        pltpu.async_copy(rows[b], out.at[pl.ds(out_row0 + g * CHUNK, CHUNK)],
                         wsem[b])

    def drain_writeback(b):
        pltpu.make_async_copy(rows[b], out.at[pl.ds(out_row0, CHUNK)],
                              wsem[b]).wait()

    for b in range(NB):
        fire_gathers(b, b)

    @pl.loop(0, N_CHUNKS, step=NB)
    def _step(g0):
        for b in range(NB):
            g = g0 + b
            drain_gathers(b)
            fire_writeback(g, b)
            nxt = g + NB

            @pl.when(nxt < N_CHUNKS)
            def _():
                drain_writeback(b)
                fire_gathers(nxt, b)

    for b in range(NB):
        drain_writeback(b)


@jax.jit
def _embedding_gather(x2d, table):
    mesh = plsc.VectorSubcoreMesh(core_axis_name="c", subcore_axis_name="s")
    run = functools.partial(
        pl.kernel,
        mesh=mesh,
        out_type=jax.ShapeDtypeStruct((B, EMBED_DIM), jnp.float32),
        scratch_types=[
            pltpu.VMEM((ROWS_PER_W, IDX_MINOR), jnp.int32),
            pltpu.VMEM((CHUNK, EMBED_DIM), jnp.float32),
            pltpu.VMEM((CHUNK, EMBED_DIM), jnp.float32),
            pltpu.SemaphoreType.DMA,
            pltpu.SemaphoreType.DMA,
            pltpu.SemaphoreType.DMA,
            pltpu.SemaphoreType.DMA,
        ],
        compiler_params=pltpu.CompilerParams(use_tc_tiling_on_sc=False),
    )(_gather_kernel)
    return run(table, x2d)


def kernel(x, embedding):
    x2d = x.reshape(IDX_ROWS, IDX_MINOR).astype(jnp.int32)
    out = _embedding_gather(x2d, embedding)
    return out.reshape(BATCH, SEQ, EMBED_DIM)


# speedup vs baseline: 2.5421x; 1.3547x over previous
import functools

import jax
import jax.numpy as jnp
from jax import lax
from jax.experimental import pallas as pl
from jax.experimental.pallas import tpu as pltpu
from jax.experimental.pallas import tpu_sc as plsc

VOCAB = 1000000
EMBED_DIM = 64
BATCH = 16384
SEQ = 50

NC = 2
NS = 16
NW = NC * NS

SEQ_PAD = 56
EMBED_PAD = 128
BPW = BATCH // NW
QS = BPW // 128
NB = 2


def _gather_kernel(table, idx, out, idx_v, rows_a, rows_b,
                   gsem_a, gsem_b, wsem_a, wsem_b):
    wid = lax.axis_index("s") * NC + lax.axis_index("c")
    b0 = wid * BPW

    rows = (rows_a, rows_b)
    gsem = (gsem_a, gsem_b)
    wsem = (wsem_a, wsem_b)

    pltpu.sync_copy(idx.at[pl.ds(0, SEQ), pl.ds(b0, BPW)], idx_v)

    def fire_gathers(s, b):
        for q in range(QS):
            pltpu.async_copy(
                table.at[idx_v.at[s, pl.ds(q * 128, 128)]],
                rows[b].at[pl.ds(q * 128, 128)],
                gsem[b],
            )

    def drain_gathers(b):
        pltpu.make_async_copy(table.at[pl.ds(0, BPW)], rows[b], gsem[b]).wait()

    def fire_writeback(s, b):
        pltpu.async_copy(rows[b],
                         out.at[pl.ds(b0, BPW), s, pl.ds(0, EMBED_DIM)],
                         wsem[b])

    def drain_writeback(b):
        pltpu.make_async_copy(rows[b],
                              out.at[pl.ds(b0, BPW), 0, pl.ds(0, EMBED_DIM)],
                              wsem[b]).wait()

    for b in range(NB):
        fire_gathers(b, b)

    @pl.loop(0, SEQ, step=NB)
    def _step(s0):
        for b in range(NB):
            s = s0 + b
            drain_gathers(b)
            fire_writeback(s, b)
            nxt = s + NB

            @pl.when(nxt < SEQ)
            def _():
                drain_writeback(b)
                fire_gathers(nxt, b)

    for b in range(NB):
        drain_writeback(b)


@jax.jit
def _embedding_gather(xt, table):
    mesh = plsc.VectorSubcoreMesh(core_axis_name="c", subcore_axis_name="s")
    run = functools.partial(
        pl.kernel,
        mesh=mesh,
        out_type=jax.ShapeDtypeStruct((BATCH, SEQ_PAD, EMBED_PAD), jnp.float32),
        scratch_types=[
            pltpu.VMEM((SEQ, BPW), jnp.int32),
            pltpu.VMEM((BPW, EMBED_DIM), jnp.float32),
            pltpu.VMEM((BPW, EMBED_DIM), jnp.float32),
            pltpu.SemaphoreType.DMA,
            pltpu.SemaphoreType.DMA,
            pltpu.SemaphoreType.DMA,
            pltpu.SemaphoreType.DMA,
        ],
        compiler_params=pltpu.CompilerParams(use_tc_tiling_on_sc=False),
    )(_gather_kernel)
    return run(table, xt)


def kernel(x, embedding):
    xt = x.T.astype(jnp.int32)
    out = _embedding_gather(xt, embedding)
    return out[:, :SEQ, :EMBED_DIM]
